# Initial kernel scaffold; baseline (speedup 1.0000x reference)
import jax
import jax.numpy as jnp
from jax.experimental import pallas as pl


def kernel(hidden_states, router_weight, router_bias, gate_up_proj, gate_up_bias, down_proj, down_bias):
    raise NotImplementedError("write your pallas kernel here")


# speedup vs baseline: 2.0451x; 2.0451x over previous
import functools

import jax
import jax.numpy as jnp
from jax.experimental import pallas as pl
from jax.experimental.pallas import tpu as pltpu

B, S, H = 1, 2048, 1024
E, K, INTER = 8, 2, 1024
ALPHA = 1.702
LIMIT = 7.0
T = B * S
CH = 256


def _moe_dense_body(x_ref, wr_ref, rb_ref, wgu_ref, gub_ref, wd_ref, db_ref,
                    out_ref, gates_ref):
    e = pl.program_id(0)

    @pl.when(e == 0)
    def _compute_gates():
        for c in range(T // CH):
            xt = x_ref[c * CH:(c + 1) * CH, :]
            logits = jax.lax.dot_general(
                xt, wr_ref[...], (((1,), (1,)), ((), ())),
                preferred_element_type=jnp.float32) + rb_ref[...]
            m = jnp.max(logits, axis=1, keepdims=True)
            p = jnp.exp(logits - m)
            s = p / jnp.sum(p, axis=1, keepdims=True)
            iota = jax.lax.broadcasted_iota(jnp.int32, (CH, E), 1)
            m1 = jnp.max(s, axis=1, keepdims=True)
            idx1 = jnp.min(jnp.where(s == m1, iota, E), axis=1, keepdims=True)
            not1 = iota != idx1
            m2 = jnp.max(jnp.where(not1, s, -jnp.inf), axis=1, keepdims=True)
            idx2 = jnp.min(jnp.where(not1 & (s == m2), iota, E), axis=1,
                           keepdims=True)
            keep = (iota == idx1) | (iota == idx2)
            gates_ref[c * CH:(c + 1) * CH, :] = jnp.where(keep, s, 0.0)

    wgu = wgu_ref[0]
    wd = wd_ref[0]
    gub = gub_ref[0]
    db = db_ref[0]
    for c in range(T // CH):
        sl = pl.ds(c * CH, CH)
        xt = x_ref[sl, :]
        gu = jnp.dot(xt, wgu, preferred_element_type=jnp.float32) + gub
        gate = jnp.minimum(gu[:, :INTER], LIMIT)
        up = jnp.clip(gu[:, INTER:], -LIMIT, LIMIT)
        glu = gate * jax.nn.sigmoid(gate * ALPHA)
        act = (up + 1.0) * glu
        y = jnp.dot(act, wd, preferred_element_type=jnp.float32) + db
        gcol = gates_ref[sl, :]
        gsel = jnp.sum(jnp.where(
            jax.lax.broadcasted_iota(jnp.int32, (CH, E), 1) == e, gcol, 0.0),
            axis=1, keepdims=True)
        contrib = y * gsel

        @pl.when(e == 0)
        def _init():
            out_ref[sl, :] = contrib

        @pl.when(e != 0)
        def _acc():
            out_ref[sl, :] = out_ref[sl, :] + contrib


def kernel(hidden_states, router_weight, router_bias, gate_up_proj,
           gate_up_bias, down_proj, down_bias):
    flat = hidden_states.reshape(T, H)
    rb2 = router_bias.reshape(1, E)

    out = pl.pallas_call(
        _moe_dense_body,
        grid=(E,),
        in_specs=[
            pl.BlockSpec((T, H), lambda e: (0, 0)),
            pl.BlockSpec((E, H), lambda e: (0, 0)),
            pl.BlockSpec((1, E), lambda e: (0, 0)),
            pl.BlockSpec((1, H, 2 * INTER), lambda e: (e, 0, 0)),
            pl.BlockSpec((1, 1, 2 * INTER), lambda e: (e, 0, 0)),
            pl.BlockSpec((1, INTER, H), lambda e: (e, 0, 0)),
            pl.BlockSpec((1, 1, H), lambda e: (e, 0, 0)),
        ],
        out_specs=pl.BlockSpec((T, H), lambda e: (0, 0)),
        out_shape=jax.ShapeDtypeStruct((T, H), jnp.float32),
        scratch_shapes=[pltpu.VMEM((T, E), jnp.float32)],
        compiler_params=pltpu.CompilerParams(
            dimension_semantics=("arbitrary",)),
    )(flat, router_weight, rb2, gate_up_proj,
      gate_up_bias.reshape(E, 1, 2 * INTER), down_proj,
      down_bias.reshape(E, 1, H))
    return out.reshape(B, S, H)
